# baseline (speedup 1.0000x reference)
import jax
import jax.numpy as jnp
from jax.experimental import pallas as pl
from jax.experimental.pallas import tpu as pltpu


def _encoder_kernel(adj_ref, x_ref, w1_ref, w2_ref, p1_ref, p2_ref,
                    w3_ref, w4_ref, p3_ref, p4_ref, w5_ref, w6_ref,
                    out_ref):
    adj = adj_ref[...]
    xb = x_ref[...]
    f32 = jnp.float32
    c0 = (((0,), (0,)), ((), ()))

    c1 = jnp.concatenate([w1_ref[...], p1_ref[...]], axis=1)
    h = jnp.dot(xb, c1, preferred_element_type=f32)
    g1 = jnp.maximum(jnp.dot(adj, h, preferred_element_type=f32), 0.0)
    g2 = jnp.dot(adj, g1, preferred_element_type=f32)
    z1 = jnp.maximum(
        jnp.dot(g2[:, :64], w2_ref[...], preferred_element_type=f32), 0.0)
    logits = jnp.dot(g2[:, 64:], p2_ref[...], preferred_element_type=f32)
    m = jnp.max(logits, axis=1, keepdims=True)
    e = jnp.exp(logits - m)
    s1 = e / jnp.sum(e, axis=1, keepdims=True)
    t = jnp.dot(adj, s1, preferred_element_type=f32)
    x2 = jax.lax.dot_general(s1, z1, c0, preferred_element_type=f32)
    adj2 = jax.lax.dot_general(s1, t, c0, preferred_element_type=f32)

    def gcn(a, hh, w):
        hw = jnp.dot(hh, w, preferred_element_type=f32)
        return jnp.maximum(jnp.dot(a, hw, preferred_element_type=f32), 0.0)

    z2 = gcn(adj2, gcn(adj2, x2, w3_ref[...]), w4_ref[...])
    sh2 = gcn(adj2, x2, p3_ref[...])
    log2 = jnp.dot(adj2, jnp.dot(sh2, p4_ref[...],
                                 preferred_element_type=f32),
                   preferred_element_type=f32)
    m2 = jnp.max(log2, axis=1, keepdims=True)
    e2 = jnp.exp(log2 - m2)
    s2 = e2 / jnp.sum(e2, axis=1, keepdims=True)
    x3 = jax.lax.dot_general(s2, z2, c0, preferred_element_type=f32)
    adj3 = jax.lax.dot_general(
        s2, jnp.dot(adj2, s2, preferred_element_type=f32), c0,
        preferred_element_type=f32)
    z3 = gcn(adj3, gcn(adj3, x3, w5_ref[...]), w6_ref[...])

    emb = jnp.concatenate(
        [jnp.max(z1, axis=0, keepdims=True),
         jnp.max(z2, axis=0, keepdims=True),
         jnp.max(z3, axis=0, keepdims=True)], axis=1)
    out_ref[0] = jnp.maximum(emb, 0.0)


def kernel(x, adj, W1, W2, P1, P2, W3, W4, P3, P4, W5, W6):
    B, N, _ = adj.shape
    f_in = x.shape[2]
    adj2d = adj.reshape(B * N, N)
    x2d = x.reshape(B * N, f_in)
    wspec = [pl.BlockSpec(w.shape, lambda b: (0, 0))
             for w in (W1, W2, P1, P2, W3, W4, P3, P4, W5, W6)]
    out = pl.pallas_call(
        _encoder_kernel,
        grid=(B,),
        in_specs=[
            pl.BlockSpec((N, N), lambda b: (b, 0)),
            pl.BlockSpec((N, f_in), lambda b: (b, 0)),
        ] + wspec,
        out_specs=pl.BlockSpec((1, 1, 192), lambda b: (b, 0, 0)),
        out_shape=jax.ShapeDtypeStruct((B, 1, 192), jnp.float32),
        compiler_params=pltpu.CompilerParams(
            dimension_semantics=("parallel",),
        ),
    )(adj2d, x2d, W1, W2, P1, P2, W3, W4, P3, P4, W5, W6)
    return out


# speedup vs baseline: 1.3572x; 1.3572x over previous
import jax
import jax.numpy as jnp
from jax.experimental import pallas as pl
from jax.experimental.pallas import tpu as pltpu


def _encoder_kernel(adj_ref, xt_ref, w1_ref, w2_ref, p1_ref, p2_ref,
                    w3_ref, w4_ref, p3_ref, p4_ref, w5_ref, w6_ref,
                    out_ref):
    adj = adj_ref[...]
    xt = xt_ref[...]
    f32 = jnp.float32
    ct = (((1,), (1,)), ((), ()))
    c0 = (((0,), (0,)), ((), ()))

    c1 = jnp.concatenate([w1_ref[...], p1_ref[...]], axis=1)
    ht = jax.lax.dot_general(c1, xt, c0, preferred_element_type=f32)
    g1t = jnp.maximum(
        jax.lax.dot_general(ht, adj, ct, preferred_element_type=f32), 0.0)
    g2t = jax.lax.dot_general(g1t, adj, ct, preferred_element_type=f32)
    z1t = jnp.maximum(
        jax.lax.dot_general(w2_ref[...], g2t[:64, :], c0,
                            preferred_element_type=f32), 0.0)
    logt = jax.lax.dot_general(p2_ref[...], g2t[64:, :], c0,
                               preferred_element_type=f32)
    m = jnp.max(logt, axis=0, keepdims=True)
    e = jnp.exp(logt - m)
    s1t = e / jnp.sum(e, axis=0, keepdims=True)
    tt = jax.lax.dot_general(s1t, adj, ct, preferred_element_type=f32)
    x2 = jax.lax.dot_general(s1t, z1t, ct, preferred_element_type=f32)
    adj2 = jax.lax.dot_general(s1t, tt, ct, preferred_element_type=f32)

    def gcn(a, h, w):
        hw = jnp.dot(h, w, preferred_element_type=f32)
        return jnp.maximum(jnp.dot(a, hw, preferred_element_type=f32), 0.0)

    z2 = gcn(adj2, gcn(adj2, x2, w3_ref[...]), w4_ref[...])
    sh2 = gcn(adj2, x2, p3_ref[...])
    log2 = jnp.dot(adj2, jnp.dot(sh2, p4_ref[...],
                                 preferred_element_type=f32),
                   preferred_element_type=f32)
    m2 = jnp.max(log2, axis=1, keepdims=True)
    e2 = jnp.exp(log2 - m2)
    s2 = e2 / jnp.sum(e2, axis=1, keepdims=True)
    x3 = jax.lax.dot_general(s2, z2, c0, preferred_element_type=f32)
    adj3 = jax.lax.dot_general(
        s2, jnp.dot(adj2, s2, preferred_element_type=f32), c0,
        preferred_element_type=f32)
    z3 = gcn(adj3, gcn(adj3, x3, w5_ref[...]), w6_ref[...])

    emb = jnp.concatenate(
        [jnp.max(z1t, axis=1, keepdims=True).T,
         jnp.max(z2, axis=0, keepdims=True),
         jnp.max(z3, axis=0, keepdims=True)], axis=1)
    out_ref[0] = jnp.maximum(emb, 0.0)


def kernel(x, adj, W1, W2, P1, P2, W3, W4, P3, P4, W5, W6):
    B, N, _ = adj.shape
    f_in = x.shape[2]
    adj2d = adj.reshape(B * N, N)
    xt2d = x.transpose(0, 2, 1).reshape(B * f_in, N)
    wspec = [pl.BlockSpec(w.shape, lambda b: (0, 0))
             for w in (W1, W2, P1, P2, W3, W4, P3, P4, W5, W6)]
    out = pl.pallas_call(
        _encoder_kernel,
        grid=(B,),
        in_specs=[
            pl.BlockSpec((N, N), lambda b: (b, 0)),
            pl.BlockSpec((f_in, N), lambda b: (b, 0)),
        ] + wspec,
        out_specs=pl.BlockSpec((1, 1, 192), lambda b: (b, 0, 0)),
        out_shape=jax.ShapeDtypeStruct((B, 1, 192), jnp.float32),
        compiler_params=pltpu.CompilerParams(
            dimension_semantics=("parallel",),
        ),
    )(adj2d, xt2d, W1, W2, P1, P2, W3, W4, P3, P4, W5, W6)
    return out


# speedup vs baseline: 1.3656x; 1.0062x over previous
import jax
import jax.numpy as jnp
from jax.experimental import pallas as pl
from jax.experimental.pallas import tpu as pltpu


def _encoder_kernel(adj_ref, xt_ref, w1_ref, w2_ref, p1_ref, p2_ref,
                    w3_ref, w4_ref, p3_ref, p4_ref, w5_ref, w6_ref,
                    out_ref):
    adj = adj_ref[...]
    xt = xt_ref[...]
    f32 = jnp.float32
    ct = (((1,), (1,)), ((), ()))
    c0 = (((0,), (0,)), ((), ()))

    c1 = jnp.concatenate([w1_ref[...], p1_ref[...]], axis=1)
    ht = jax.lax.dot_general(c1, xt, c0, preferred_element_type=f32)
    g1t = jnp.maximum(
        jax.lax.dot_general(ht, adj, ct, preferred_element_type=f32), 0.0)
    g2t = jax.lax.dot_general(g1t, adj, ct, preferred_element_type=f32)
    z1t = jnp.maximum(
        jax.lax.dot_general(w2_ref[...], g2t[:64, :], c0,
                            preferred_element_type=f32), 0.0)
    logt = jax.lax.dot_general(p2_ref[...], g2t[64:, :], c0,
                               preferred_element_type=f32)
    m = jnp.max(logt, axis=0, keepdims=True)
    e = jnp.exp(logt - m)
    s1t = e / jnp.sum(e, axis=0, keepdims=True)
    tt = jax.lax.dot_general(s1t, adj, ct, preferred_element_type=f32)
    x2 = jax.lax.dot_general(s1t, z1t, ct, preferred_element_type=f32)
    adj2 = jax.lax.dot_general(s1t, tt, ct, preferred_element_type=f32)

    def gcn(a, h, w):
        hw = jnp.dot(h, w, preferred_element_type=f32)
        return jnp.maximum(jnp.dot(a, hw, preferred_element_type=f32), 0.0)

    z2 = gcn(adj2, gcn(adj2, x2, w3_ref[...]), w4_ref[...])
    sh2 = gcn(adj2, x2, p3_ref[...])
    log2 = jnp.dot(adj2, jnp.dot(sh2, p4_ref[...],
                                 preferred_element_type=f32),
                   preferred_element_type=f32)
    m2 = jnp.max(log2, axis=1, keepdims=True)
    e2 = jnp.exp(log2 - m2)
    s2 = e2 / jnp.sum(e2, axis=1, keepdims=True)
    x3 = jax.lax.dot_general(s2, z2, c0, preferred_element_type=f32)
    adj3 = jax.lax.dot_general(
        s2, jnp.dot(adj2, s2, preferred_element_type=f32), c0,
        preferred_element_type=f32)
    z3 = gcn(adj3, gcn(adj3, x3, w5_ref[...]), w6_ref[...])

    emb = jnp.concatenate(
        [jnp.max(z1t, axis=1, keepdims=True).T,
         jnp.max(z2, axis=0, keepdims=True),
         jnp.max(z3, axis=0, keepdims=True)], axis=1)
    out_ref[0] = jnp.maximum(emb, 0.0)


def kernel(x, adj, W1, W2, P1, P2, W3, W4, P3, P4, W5, W6):
    B, N, _ = adj.shape
    f_in = x.shape[2]
    adj2d = adj.reshape(B * N, N)
    xt2d = x.transpose(0, 2, 1).reshape(B * f_in, N)
    wspec = [pl.BlockSpec(w.shape, lambda b: (0, 0))
             for w in (W1, W2, P1, P2, W3, W4, P3, P4, W5, W6)]
    out = pl.pallas_call(
        _encoder_kernel,
        grid=(B,),
        in_specs=[
            pl.BlockSpec((N, N), lambda b: (b, 0)),
            pl.BlockSpec((f_in, N), lambda b: (b, 0)),
        ] + wspec,
        out_specs=pl.BlockSpec((1, 1, 192), lambda b: (b, 0, 0)),
        out_shape=jax.ShapeDtypeStruct((B, 1, 192), jnp.float32),
        compiler_params=pltpu.CompilerParams(
            dimension_semantics=("parallel",),
            vmem_limit_bytes=100 * 1024 * 1024,
        ),
    )(adj2d, xt2d, W1, W2, P1, P2, W3, W4, P3, P4, W5, W6)
    return out
